# Initial kernel scaffold; baseline (speedup 1.0000x reference)
import jax
import jax.numpy as jnp
from jax.experimental import pallas as pl


def kernel(costh, label, omega):
    raise NotImplementedError("write your pallas kernel here")


# speedup vs baseline: 665.2988x; 665.2988x over previous
import functools

import jax
import jax.numpy as jnp
from jax.experimental import pallas as pl
from jax.experimental.pallas import tpu as pltpu

ALPHA = 40.0
BETA = 0.25
GAMMA = 0.75


def _body(costh_ref, label_ref, omega_ref, out_ref, acc_ref, *, B, C):
    i = pl.program_id(0)
    n = pl.num_programs(0)
    om = omega_ref[0]
    x = costh_ref[...]
    z = ALPHA * (om - x)
    s = 1.0 / (1.0 + jnp.exp(-z))
    lbl = label_ref[...]
    cols = jax.lax.broadcasted_iota(jnp.int32, s.shape, 1)
    pos = jnp.sum(jnp.where(cols == lbl, s, 0.0))
    tot = jnp.sum(s)

    @pl.when(i == 0)
    def _():
        acc_ref[0] = 0.0
        acc_ref[1] = 0.0

    acc_ref[0] += tot
    acc_ref[1] += pos

    @pl.when(i == n - 1)
    def _():
        s_all = acc_ref[0]
        s_pos = acc_ref[1]
        pfa = GAMMA * (B - s_pos) / B
        pmiss = BETA * (s_all - s_pos) / (B * (C - 1))
        out_ref[0] = pfa + pmiss


def kernel(costh, label, omega):
    B, C = costh.shape
    BR = 1024
    label2d = label.astype(jnp.int32).reshape(B, 1)
    omega1 = omega.astype(jnp.float32).reshape(1)
    out = pl.pallas_call(
        functools.partial(_body, B=B, C=C),
        grid=(B // BR,),
        in_specs=[
            pl.BlockSpec((BR, C), lambda i: (i, 0)),
            pl.BlockSpec((BR, 1), lambda i: (i, 0)),
            pl.BlockSpec(memory_space=pltpu.SMEM),
        ],
        out_specs=pl.BlockSpec(memory_space=pltpu.SMEM),
        out_shape=jax.ShapeDtypeStruct((1,), jnp.float32),
        scratch_shapes=[pltpu.SMEM((2,), jnp.float32)],
    )(costh, label2d, omega1)
    return out[0]


# speedup vs baseline: 681.1998x; 1.0239x over previous
import functools

import jax
import jax.numpy as jnp
from jax.experimental import pallas as pl
from jax.experimental.pallas import tpu as pltpu

ALPHA = 40.0
BETA = 0.25
GAMMA = 0.75
HALPHA = ALPHA * 0.5


def _body(costh_ref, label_ref, omega_ref, out_ref, acc_ref, *, B, C):
    i = pl.program_id(0)
    n = pl.num_programs(0)
    c = HALPHA * omega_ref[0]
    x = costh_ref[...]
    t = jnp.tanh(c - HALPHA * x)
    lbl = label_ref[...]
    cols = jax.lax.broadcasted_iota(jnp.int32, t.shape, 1)
    pos = jnp.sum(jnp.where(cols == lbl, t, 0.0))
    tot = jnp.sum(t)

    @pl.when(i == 0)
    def _():
        acc_ref[0] = 0.0
        acc_ref[1] = 0.0

    acc_ref[0] += tot
    acc_ref[1] += pos

    @pl.when(i == n - 1)
    def _():
        t_all = acc_ref[0]
        t_pos = acc_ref[1]
        pfa = GAMMA * 0.5 * (1.0 - t_pos / B)
        pmiss = BETA * 0.5 * ((B * (C - 1) + t_all - t_pos) / (B * (C - 1)))
        out_ref[0] = pfa + pmiss


def kernel(costh, label, omega):
    B, C = costh.shape
    BR = 1024
    label2d = label.astype(jnp.int32).reshape(B, 1)
    omega1 = omega.astype(jnp.float32).reshape(1)
    out = pl.pallas_call(
        functools.partial(_body, B=B, C=C),
        grid=(B // BR,),
        in_specs=[
            pl.BlockSpec((BR, C), lambda i: (i, 0)),
            pl.BlockSpec((BR, 1), lambda i: (i, 0)),
            pl.BlockSpec(memory_space=pltpu.SMEM),
        ],
        out_specs=pl.BlockSpec(memory_space=pltpu.SMEM),
        out_shape=jax.ShapeDtypeStruct((1,), jnp.float32),
        scratch_shapes=[pltpu.SMEM((2,), jnp.float32)],
    )(costh, label2d, omega1)
    return out[0]
